# baseline (speedup 1.0000x reference)
import jax
import jax.numpy as jnp
from jax.experimental import pallas as pl
from jax.experimental.pallas import tpu as pltpu

TOPK = 2
NEXP = 8
CHUNK = 1024
NBUF = 3


def _gate_kernel(x_ref, w_ref, idx_ref, wgt_ref, buf_ref, sem_ref):
    n = x_ref.shape[0]
    nchunk = n // CHUNK
    w = w_ref[...]

    def start_copy(c, s):
        pltpu.make_async_copy(
            x_ref.at[pl.ds(c * CHUNK, CHUNK), :], buf_ref.at[s], sem_ref.at[s]
        ).start()

    def wait_copy(c, s):
        pltpu.make_async_copy(
            x_ref.at[pl.ds(c * CHUNK, CHUNK), :], buf_ref.at[s], sem_ref.at[s]
        ).wait()

    for k in range(NBUF):
        start_copy(k, k)

    def body(i, carry):
        s = jax.lax.rem(i, NBUF)
        wait_copy(i, s)
        x = buf_ref[s]
        logits = jax.lax.dot_general(
            w, x, (((1,), (1,)), ((), ())), preferred_element_type=jnp.float32
        )
        iota = jax.lax.broadcasted_iota(jnp.int32, (NEXP, CHUNK), 0)
        m1 = jnp.max(logits, axis=0, keepdims=True)
        idx1 = jnp.min(
            jnp.where(logits == m1, iota, NEXP), axis=0, keepdims=True
        )
        masked = jnp.where(iota == idx1, -jnp.inf, logits)
        m2 = jnp.max(masked, axis=0, keepdims=True)
        idx2 = jnp.min(
            jnp.where(masked == m2, iota, NEXP), axis=0, keepdims=True
        )
        z = jnp.sum(jnp.exp(logits - m1), axis=0, keepdims=True)
        w1 = 1.0 / z
        w2 = jnp.exp(m2 - m1) / z
        idx_ref[:, pl.ds(i * CHUNK, CHUNK)] = jnp.concatenate(
            [idx1, idx2], axis=0
        )
        wgt_ref[:, pl.ds(i * CHUNK, CHUNK)] = jnp.concatenate([w1, w2], axis=0)

        nxt = i + NBUF

        @pl.when(nxt < nchunk)
        def _():
            start_copy(nxt, s)

        return carry

    jax.lax.fori_loop(0, nchunk, body, 0)


@jax.jit
def kernel(hidden_states, weight):
    bsz, seq_len, h = hidden_states.shape
    n = bsz * seq_len
    x = hidden_states.reshape(n, h)

    idx_t, wgt_t = pl.pallas_call(
        _gate_kernel,
        in_specs=[
            pl.BlockSpec(memory_space=pl.ANY),
            pl.BlockSpec(memory_space=pltpu.VMEM),
        ],
        out_specs=[
            pl.BlockSpec(memory_space=pltpu.VMEM),
            pl.BlockSpec(memory_space=pltpu.VMEM),
        ],
        out_shape=[
            jax.ShapeDtypeStruct((TOPK, n), jnp.int32),
            jax.ShapeDtypeStruct((TOPK, n), jnp.float32),
        ],
        scratch_shapes=[
            pltpu.VMEM((NBUF, CHUNK, h), jnp.float32),
            pltpu.SemaphoreType.DMA((NBUF,)),
        ],
    )(x, weight)
    return idx_t.T, wgt_t.T
